# baseline (speedup 1.0000x reference)
import functools
import numpy as np
import jax
import jax.numpy as jnp
from jax import lax
from jax.experimental import pallas as pl
from jax.experimental.pallas import tpu as pltpu
from jax.experimental.pallas import tpu_sc as plsc

_B = 2
_S = 8192
_D = 768
_H = 12
_DK = 64
_DV = 64
_EXP = 4
_BKT = 64
_NC = _S // _BKT
_NHASH = _NC
_SB = 512
_CPB = 16
_NCB = _NC // _CPB


def _rot_const(seed):
    rng = np.random.default_rng(seed)
    nb = _S // _BKT
    r = rng.standard_normal((1, _H, _DK, nb // 2))
    return jnp.asarray(r[0], dtype=jnp.float32)


def _qkv_hash_body(xq_ref, xkv_ref, wq_ref, wk_ref, wv_ref, rot_ref,
                   q_ref, kv_ref, bq_ref, bk_ref):
    xq = xq_ref[0]
    xkv = xkv_ref[0]
    q = jnp.dot(xq, wq_ref[...], preferred_element_type=jnp.float32)
    k = jnp.dot(xkv, wk_ref[...], preferred_element_type=jnp.float32)
    v = jnp.dot(xkv, wv_ref[...], preferred_element_type=jnp.float32)
    z = jnp.zeros((_SB, _DK), jnp.float32)
    qcols, kvcols = [], []
    for h in range(_H):
        qcols += [q[:, h * _DK:(h + 1) * _DK], z]
        kvcols += [k[:, h * _DK:(h + 1) * _DK], v[:, h * _DV:(h + 1) * _DV]]
    q_ref[0] = jnp.concatenate(qcols, axis=1)
    kv_ref[0] = jnp.concatenate(kvcols, axis=1)

    def buckets(mat):
        cols = []
        for h in range(_H):
            mh = mat[:, h * _DK:(h + 1) * _DK]
            rq = jnp.dot(mh, rot_ref[h], preferred_element_type=jnp.float32)
            sc = jnp.concatenate([rq, -rq], axis=-1)
            m = jnp.max(sc, axis=-1, keepdims=True)
            lane = jax.lax.broadcasted_iota(jnp.int32, sc.shape, 1)
            idx = jnp.min(jnp.where(sc >= m, lane, _NHASH), axis=-1)
            cols.append(idx.reshape(_SB, 1))
        cols.append(jnp.zeros((_SB, 128 - _H), jnp.int32))
        return jnp.concatenate(cols, axis=-1)

    bq_ref[0] = buckets(q)
    bk_ref[0] = buckets(k)


def _qkv_hash(xq, xkv, wq, wk, wv, rot):
    nsb = _S // _SB
    grid = (_B, nsb)
    io_spec = pl.BlockSpec((1, _SB, _D), lambda b, s: (b, s, 0))
    w_spec = pl.BlockSpec((_D, _H * _DK), lambda b, s: (0, 0))
    p_spec = pl.BlockSpec((1, _SB, _H * 128), lambda b, s: (b, s, 0))
    b_spec = pl.BlockSpec((1, _SB, 128), lambda b, s: (b, s, 0))
    out = pl.pallas_call(
        _qkv_hash_body,
        grid=grid,
        in_specs=[io_spec, io_spec, w_spec, w_spec, w_spec,
                  pl.BlockSpec((_H, _DK, 64), lambda b, s: (0, 0, 0))],
        out_specs=[p_spec, p_spec, b_spec, b_spec],
        out_shape=[
            jax.ShapeDtypeStruct((_B, _S, _H * 128), jnp.float32),
            jax.ShapeDtypeStruct((_B, _S, _H * 128), jnp.float32),
            jax.ShapeDtypeStruct((_B, _S, 128), jnp.int32),
            jax.ShapeDtypeStruct((_B, _S, 128), jnp.int32),
        ],
    )(xq, xkv, wq, wk, wv, rot)
    return out


_RB = 256
_NW = 32
_CH = 128
_NCHUNK = _B * _S * _H // (_NW * _CH)


def _hist_body(bq_ref, bk_ref, hq_ref, hk_ref):
    s = pl.program_id(1)

    def hist(blk):
        lane = jax.lax.broadcasted_iota(jnp.int32, (_RB, 128), 1)
        rows = []
        for h in range(_H):
            oh = (blk[:, h:h + 1] == lane).astype(jnp.float32)
            rows.append(jnp.sum(oh, axis=0, keepdims=True))
        rows.append(jnp.zeros((16 - _H, 128), jnp.float32))
        return jnp.concatenate(rows, axis=0)

    hq = hist(bq_ref[0])
    hk = hist(bk_ref[0])

    @pl.when(s == 0)
    def _():
        hq_ref[0] = hq
        hk_ref[0] = hk

    @pl.when(s != 0)
    def _():
        hq_ref[0] += hq
        hk_ref[0] += hk


def _hist2(bq, bk):
    grid = (_B, _S // _RB)
    bspec = pl.BlockSpec((1, _RB, 128), lambda b, s: (b, s, 0))
    hspec = pl.BlockSpec((1, 16, 128), lambda b, s: (b, 0, 0))
    return pl.pallas_call(
        _hist_body,
        grid=grid,
        in_specs=[bspec, bspec],
        out_specs=[hspec, hspec],
        out_shape=[jax.ShapeDtypeStruct((_B, 16, 128), jnp.float32)] * 2,
    )(bq, bk)


def _rank_body(bq_ref, bk_ref, hq_ref, hk_ref, dq_ref, dk_ref, cq_ref, ck_ref):
    b = pl.program_id(0)
    s = pl.program_id(1)

    @pl.when(s == 0)
    def _():
        cq_ref[...] = jnp.zeros((16, 128), jnp.float32)
        ck_ref[...] = jnp.zeros((16, 128), jnp.float32)

    i128 = jax.lax.broadcasted_iota(jnp.int32, (128, 128), 0)
    j128 = jax.lax.broadcasted_iota(jnp.int32, (128, 128), 1)
    upper = (i128 < j128).astype(jnp.float32)
    ib = jax.lax.broadcasted_iota(jnp.int32, (_RB, _RB), 0)
    jb = jax.lax.broadcasted_iota(jnp.int32, (_RB, _RB), 1)
    lower = (ib > jb).astype(jnp.float32)
    lane = jax.lax.broadcasted_iota(jnp.int32, (_RB, 128), 1)

    def rank(blk, h_ref, c_ref):
        cum = jnp.dot(h_ref[0], upper, preferred_element_type=jnp.float32)
        cols = []
        for h in range(_H):
            oh = (blk[:, h:h + 1] == lane).astype(jnp.float32)
            p = jnp.dot(lower, oh, preferred_element_type=jnp.float32)
            base = cum[h:h + 1, :] + c_ref[h:h + 1, :]
            pos = jnp.sum((p + base) * oh, axis=1)
            c_ref[h:h + 1, :] += jnp.sum(oh, axis=0, keepdims=True)
            dst = pos.astype(jnp.int32) + (b * _H + h) * _S
            cols.append(dst.reshape(_RB, 1))
        cols.append(jnp.zeros((_RB, 128 - _H), jnp.int32))
        return jnp.concatenate(cols, axis=1)

    dq_ref[0] = rank(bq_ref[0], hq_ref, cq_ref)
    dk_ref[0] = rank(bk_ref[0], hk_ref, ck_ref)


def _rank2(bq, bk, hq, hk):
    grid = (_B, _S // _RB)
    bspec = pl.BlockSpec((1, _RB, 128), lambda b, s: (b, s, 0))
    hspec = pl.BlockSpec((1, 16, 128), lambda b, s: (b, 0, 0))
    return pl.pallas_call(
        _rank_body,
        grid=grid,
        in_specs=[bspec, bspec, hspec, hspec],
        out_specs=[bspec, bspec],
        out_shape=[jax.ShapeDtypeStruct((_B, _S, 128), jnp.int32)] * 2,
        scratch_shapes=[pltpu.VMEM((16, 128), jnp.float32)] * 2,
    )(bq, bk, hq, hk)


def _sc_scatter2(qf, kvf, idxq, idxk):
    nrows = _B * _S * _H
    mesh = plsc.VectorSubcoreMesh(core_axis_name="c", subcore_axis_name="s")

    @functools.partial(
        pl.kernel, mesh=mesh,
        out_type=[jax.ShapeDtypeStruct((nrows, 128), jnp.float32)] * 2,
        scratch_types=[
            pltpu.VMEM((_CH,), jnp.int32),
            pltpu.VMEM((_CH,), jnp.int32),
            pltpu.VMEM((_CH, 128), jnp.float32),
            pltpu.VMEM((_CH, 128), jnp.float32),
            pltpu.SemaphoreType.DMA,
        ],
    )
    def k(q_hbm, kv_hbm, iq_hbm, ik_hbm, qs_hbm, kvs_hbm,
          iq_v, ik_v, qrows, kvrows, sem):
        wid = lax.axis_index("s") * 2 + lax.axis_index("c")

        def body(j, _):
            base = pl.multiple_of((wid * _NCHUNK + j) * _CH, _CH)
            pltpu.sync_copy(iq_hbm.at[wid, j], iq_v)
            pltpu.sync_copy(ik_hbm.at[wid, j], ik_v)
            pltpu.sync_copy(q_hbm.at[pl.ds(base, _CH)], qrows)
            pltpu.sync_copy(kv_hbm.at[pl.ds(base, _CH)], kvrows)
            pltpu.async_copy(qrows, qs_hbm.at[iq_v], sem).wait()
            pltpu.async_copy(kvrows, kvs_hbm.at[ik_v], sem).wait()
            return 0

        lax.fori_loop(0, _NCHUNK, body, 0)

    return k(qf, kvf, idxq, idxk)


def _sc_gather(ocf, idxq):
    nrows = _B * _S * _H
    mesh = plsc.VectorSubcoreMesh(core_axis_name="c", subcore_axis_name="s")

    @functools.partial(
        pl.kernel, mesh=mesh,
        out_type=jax.ShapeDtypeStruct((nrows, 128), jnp.float32),
        scratch_types=[
            pltpu.VMEM((_CH,), jnp.int32),
            pltpu.VMEM((_CH, 128), jnp.float32),
            pltpu.SemaphoreType.DMA,
        ],
    )
    def k(oc_hbm, iq_hbm, out_hbm, iq_v, rows, sem):
        wid = lax.axis_index("s") * 2 + lax.axis_index("c")

        def body(j, _):
            base = pl.multiple_of((wid * _NCHUNK + j) * _CH, _CH)
            pltpu.sync_copy(iq_hbm.at[wid, j], iq_v)
            pltpu.async_copy(oc_hbm.at[iq_v], rows, sem).wait()
            pltpu.sync_copy(rows, out_hbm.at[pl.ds(base, _CH)])
            return 0

        lax.fori_loop(0, _NCHUNK, body, 0)

    return k(ocf, idxq)


def _attn_body(qs_ref, kvs_ref, kvp_ref, o_ref):
    scale = 1.0 / np.sqrt(_DK).astype(np.float32)
    for i in range(_CPB):
        qc = qs_ref[0, 0, i * _BKT:(i + 1) * _BKT, :_DK]
        kvc = kvs_ref[0, 0, i * _BKT:(i + 1) * _BKT, :]
        if i == 0:
            kvp = kvp_ref[0, 0, (_CPB - 1) * _BKT:, :]
        else:
            kvp = kvs_ref[0, 0, (i - 1) * _BKT:i * _BKT, :]
        ke = jnp.concatenate([kvc[:, :_DK], kvp[:, :_DK]], axis=0)
        ve = jnp.concatenate([kvc[:, _DK:], kvp[:, _DK:]], axis=0)
        s = jnp.dot(qc, ke.T, preferred_element_type=jnp.float32) * scale
        m = jnp.max(s, axis=-1, keepdims=True)
        e = jnp.exp(s - m)
        a = e / jnp.sum(e, axis=-1, keepdims=True)
        o = jnp.dot(a, ve, preferred_element_type=jnp.float32)
        o_ref[0, 0, i * _BKT:(i + 1) * _BKT, :] = jnp.concatenate(
            [o, jnp.zeros((_BKT, 128 - _DV), jnp.float32)], axis=1)


def _chunk_attn(qs, kvs):
    grid = (_B, _H, _NCB)
    blk = _CPB * _BKT
    spec = pl.BlockSpec((1, 1, blk, 128), lambda b, h, c: (b, h, c, 0))
    prev = pl.BlockSpec((1, 1, blk, 128),
                        lambda b, h, c: (b, h, (c + _NCB - 1) % _NCB, 0))
    return pl.pallas_call(
        _attn_body,
        grid=grid,
        in_specs=[spec, spec, prev],
        out_specs=spec,
        out_shape=jax.ShapeDtypeStruct((_B, _H, _S, 128), jnp.float32),
    )(qs, kvs, kvs)


def _ln(x, g, b):
    m = jnp.mean(x, axis=-1, keepdims=True)
    xc = x - m
    v = jnp.mean(xc * xc, axis=-1, keepdims=True)
    return xc * jax.lax.rsqrt(v + 1e-6) * g + b


def _proj_ln_body(o_ref, x_ref, wo_ref, g_ref, b_ref, out_ref):
    o = jnp.dot(o_ref[0], wo_ref[...], preferred_element_type=jnp.float32)
    out_ref[0] = _ln(x_ref[0] + o, g_ref[...], b_ref[...])


def _proj_ln(o, x, wo, g, b):
    grid = (_B, _S // _SB)
    dk = o.shape[-1]
    o_spec = pl.BlockSpec((1, _SB, dk), lambda bb, s: (bb, s, 0))
    io_spec = pl.BlockSpec((1, _SB, _D), lambda bb, s: (bb, s, 0))
    return pl.pallas_call(
        _proj_ln_body,
        grid=grid,
        in_specs=[o_spec, io_spec,
                  pl.BlockSpec((dk, _D), lambda bb, s: (0, 0)),
                  pl.BlockSpec((_D,), lambda bb, s: (0,)),
                  pl.BlockSpec((_D,), lambda bb, s: (0,))],
        out_specs=io_spec,
        out_shape=jax.ShapeDtypeStruct((_B, _S, _D), jnp.float32),
    )(o, x, wo, g, b)


def _ffn_body(x_ref, w1_ref, b1_ref, w2_ref, b2_ref, g_ref, b_ref, out_ref):
    x = x_ref[0]
    h = jnp.dot(x, w1_ref[...], preferred_element_type=jnp.float32) + b1_ref[...]
    h = jnp.maximum(h, 0.0)
    y = jnp.dot(h, w2_ref[...], preferred_element_type=jnp.float32) + b2_ref[...]
    out_ref[0] = _ln(x + y, g_ref[...], b_ref[...])


def _ffn(x, p):
    grid = (_B, _S // _SB)
    io_spec = pl.BlockSpec((1, _SB, _D), lambda bb, s: (bb, s, 0))
    return pl.pallas_call(
        _ffn_body,
        grid=grid,
        in_specs=[io_spec,
                  pl.BlockSpec((_D, _EXP * _D), lambda bb, s: (0, 0)),
                  pl.BlockSpec((_EXP * _D,), lambda bb, s: (0,)),
                  pl.BlockSpec((_EXP * _D, _D), lambda bb, s: (0, 0)),
                  pl.BlockSpec((_D,), lambda bb, s: (0,)),
                  pl.BlockSpec((_D,), lambda bb, s: (0,)),
                  pl.BlockSpec((_D,), lambda bb, s: (0,))],
        out_specs=io_spec,
        out_shape=jax.ShapeDtypeStruct((_B, _S, _D), jnp.float32),
    )(x, p['W1'], p['b1'], p['W2'], p['b2'], p['ln_g'], p['ln_b'])


def _lsh_attn(qin, kvin, p, rot):
    qp, kv, bq, bk = _qkv_hash(qin, kvin, p['Wq'], p['Wk'], p['Wv'], rot)
    hq, hk = _hist2(bq, bk)
    dq, dk = _rank2(bq, bk, hq, hk)
    idxq = dq[:, :, :_H].reshape(_NW, _NCHUNK, _CH)
    idxk = dk[:, :, :_H].reshape(_NW, _NCHUNK, _CH)
    qf = qp.reshape(_B * _S * _H, 128)
    kvf = kv.reshape(_B * _S * _H, 128)
    qs, kvs = _sc_scatter2(qf, kvf, idxq, idxk)
    qs = qs.reshape(_B, _H, _S, 128)
    kvs = kvs.reshape(_B, _H, _S, 128)
    oc = _chunk_attn(qs, kvs)
    o = _sc_gather(oc.reshape(_B * _H * _S, 128), idxq)
    o = o.reshape(_B, _S, _H * 128)
    wo_pad = jnp.zeros((_H, 128, _D), jnp.float32)
    wo_pad = wo_pad.at[:, :_DV, :].set(p['Wo'].reshape(_H, _DV, _D))
    wo_pad = wo_pad.reshape(_H * 128, _D)
    return _proj_ln(o, qin, wo_pad, p['ln_g'], p['ln_b'])


def kernel(decoder_input, encoder_output, params):
    rot1 = _rot_const(1)
    rot2 = _rot_const(2)
    x = _lsh_attn(decoder_input, decoder_input, params['self'], rot1)
    x = _lsh_attn(x, encoder_output, params['enc'], rot2)
    return _ffn(x, params['ff'])


# speedup vs baseline: 1.0355x; 1.0355x over previous
import functools
import numpy as np
import jax
import jax.numpy as jnp
from jax import lax
from jax.experimental import pallas as pl
from jax.experimental.pallas import tpu as pltpu
from jax.experimental.pallas import tpu_sc as plsc

_B = 2
_S = 8192
_D = 768
_H = 12
_DK = 64
_DV = 64
_EXP = 4
_BKT = 64
_NC = _S // _BKT
_NHASH = _NC
_SB = 512
_CPB = 16
_NCB = _NC // _CPB


def _rot_const(seed):
    rng = np.random.default_rng(seed)
    nb = _S // _BKT
    r = rng.standard_normal((1, _H, _DK, nb // 2))
    return jnp.asarray(r[0], dtype=jnp.float32)


def _qkv_hash_body(xq_ref, xkv_ref, wq_ref, wkv_ref, rot_ref,
                   q_ref, kv_ref, bq_ref, bk_ref):
    xq = xq_ref[0]
    xkv = xkv_ref[0]
    q = jnp.dot(xq, wq_ref[...], preferred_element_type=jnp.float32)
    kv = jnp.dot(xkv, wkv_ref[...], preferred_element_type=jnp.float32)
    q_ref[0] = q
    kv_ref[0] = kv

    def buckets(mat, off):
        cols = []
        for h in range(_H):
            mh = mat[:, h * 128 + off:h * 128 + off + _DK]
            rq = jnp.dot(mh, rot_ref[h], preferred_element_type=jnp.float32)
            sc = jnp.concatenate([rq, -rq], axis=-1)
            m = jnp.max(sc, axis=-1, keepdims=True)
            lane = jax.lax.broadcasted_iota(jnp.int32, sc.shape, 1)
            idx = jnp.min(jnp.where(sc >= m, lane, _NHASH), axis=-1)
            cols.append(idx.reshape(_SB, 1))
        cols.append(jnp.zeros((_SB, 128 - _H), jnp.int32))
        return jnp.concatenate(cols, axis=-1)

    bq_ref[0] = buckets(q, 0)
    bk_ref[0] = buckets(kv, 0)


def _qkv_hash(xq, xkv, wq, wk, wv, rot):
    nsb = _S // _SB
    grid = (_B, nsb)
    io_spec = pl.BlockSpec((1, _SB, _D), lambda b, s: (b, s, 0))
    w_spec = pl.BlockSpec((_D, _H * 128), lambda b, s: (0, 0))
    p_spec = pl.BlockSpec((1, _SB, _H * 128), lambda b, s: (b, s, 0))
    b_spec = pl.BlockSpec((1, _SB, 128), lambda b, s: (b, s, 0))
    wq_pad = jnp.zeros((_D, _H, 128), jnp.float32)
    wq_pad = wq_pad.at[:, :, :_DK].set(wq.reshape(_D, _H, _DK))
    wkv = jnp.zeros((_D, _H, 128), jnp.float32)
    wkv = wkv.at[:, :, :_DK].set(wk.reshape(_D, _H, _DK))
    wkv = wkv.at[:, :, _DK:].set(wv.reshape(_D, _H, _DV))
    out = pl.pallas_call(
        _qkv_hash_body,
        grid=grid,
        in_specs=[io_spec, io_spec, w_spec, w_spec,
                  pl.BlockSpec((_H, _DK, 64), lambda b, s: (0, 0, 0))],
        out_specs=[p_spec, p_spec, b_spec, b_spec],
        out_shape=[
            jax.ShapeDtypeStruct((_B, _S, _H * 128), jnp.float32),
            jax.ShapeDtypeStruct((_B, _S, _H * 128), jnp.float32),
            jax.ShapeDtypeStruct((_B, _S, 128), jnp.int32),
            jax.ShapeDtypeStruct((_B, _S, 128), jnp.int32),
        ],
    )(xq, xkv, wq_pad.reshape(_D, _H * 128), wkv.reshape(_D, _H * 128), rot)
    return out


_RB = 256
_NW = 32
_CH = 128
_NCHUNK = _B * _S * _H // (_NW * _CH)


def _hist_body(bq_ref, bk_ref, hq_ref, hk_ref):
    s = pl.program_id(1)

    def hist(blk):
        lane = jax.lax.broadcasted_iota(jnp.int32, (_RB, 128), 1)
        rows = []
        for h in range(_H):
            oh = (blk[:, h:h + 1] == lane).astype(jnp.float32)
            rows.append(jnp.sum(oh, axis=0, keepdims=True))
        rows.append(jnp.zeros((16 - _H, 128), jnp.float32))
        return jnp.concatenate(rows, axis=0)

    hq = hist(bq_ref[0])
    hk = hist(bk_ref[0])

    @pl.when(s == 0)
    def _():
        hq_ref[0] = hq
        hk_ref[0] = hk

    @pl.when(s != 0)
    def _():
        hq_ref[0] += hq
        hk_ref[0] += hk


def _hist2(bq, bk):
    grid = (_B, _S // _RB)
    bspec = pl.BlockSpec((1, _RB, 128), lambda b, s: (b, s, 0))
    hspec = pl.BlockSpec((1, 16, 128), lambda b, s: (b, 0, 0))
    return pl.pallas_call(
        _hist_body,
        grid=grid,
        in_specs=[bspec, bspec],
        out_specs=[hspec, hspec],
        out_shape=[jax.ShapeDtypeStruct((_B, 16, 128), jnp.float32)] * 2,
    )(bq, bk)


def _rank_body(bq_ref, bk_ref, hq_ref, hk_ref, dq_ref, dk_ref, cq_ref, ck_ref):
    b = pl.program_id(0)
    s = pl.program_id(1)

    @pl.when(s == 0)
    def _():
        cq_ref[...] = jnp.zeros((16, 128), jnp.float32)
        ck_ref[...] = jnp.zeros((16, 128), jnp.float32)

    i128 = jax.lax.broadcasted_iota(jnp.int32, (128, 128), 0)
    j128 = jax.lax.broadcasted_iota(jnp.int32, (128, 128), 1)
    upper = (i128 < j128).astype(jnp.float32)
    ib = jax.lax.broadcasted_iota(jnp.int32, (_RB, _RB), 0)
    jb = jax.lax.broadcasted_iota(jnp.int32, (_RB, _RB), 1)
    lower = (ib > jb).astype(jnp.bfloat16)
    lane = jax.lax.broadcasted_iota(jnp.int32, (_RB, 128), 1)

    def rank(blk, h_ref, c_ref):
        cum = jnp.dot(h_ref[0], upper, preferred_element_type=jnp.float32)
        cols = []
        for h in range(_H):
            ohb = (blk[:, h:h + 1] == lane).astype(jnp.bfloat16)
            oh = ohb.astype(jnp.float32)
            p = jnp.dot(lower, ohb, preferred_element_type=jnp.float32)
            base = cum[h:h + 1, :] + c_ref[h:h + 1, :]
            pos = jnp.sum((p + base) * oh, axis=1)
            c_ref[h:h + 1, :] += jnp.sum(oh, axis=0, keepdims=True)
            dst = pos.astype(jnp.int32) + (b * _H + h) * _S
            cols.append(dst.reshape(_RB, 1))
        cols.append(jnp.zeros((_RB, 128 - _H), jnp.int32))
        return jnp.concatenate(cols, axis=1)

    dq_ref[0] = rank(bq_ref[0], hq_ref, cq_ref)
    dk_ref[0] = rank(bk_ref[0], hk_ref, ck_ref)


def _rank2(bq, bk, hq, hk):
    grid = (_B, _S // _RB)
    bspec = pl.BlockSpec((1, _RB, 128), lambda b, s: (b, s, 0))
    hspec = pl.BlockSpec((1, 16, 128), lambda b, s: (b, 0, 0))
    return pl.pallas_call(
        _rank_body,
        grid=grid,
        in_specs=[bspec, bspec, hspec, hspec],
        out_specs=[bspec, bspec],
        out_shape=[jax.ShapeDtypeStruct((_B, _S, 128), jnp.int32)] * 2,
        scratch_shapes=[pltpu.VMEM((16, 128), jnp.float32)] * 2,
    )(bq, bk, hq, hk)


def _sc_scatter2(qf, kvf, idxq, idxk):
    nrows = _B * _S * _H
    mesh = plsc.VectorSubcoreMesh(core_axis_name="c", subcore_axis_name="s")

    @functools.partial(
        pl.kernel, mesh=mesh,
        out_type=[jax.ShapeDtypeStruct((nrows, 128), jnp.float32)] * 2,
        scratch_types=[
            pltpu.VMEM((2, _CH), jnp.int32),
            pltpu.VMEM((2, _CH), jnp.int32),
            pltpu.VMEM((2, _CH, 128), jnp.float32),
            pltpu.VMEM((2, _CH, 128), jnp.float32),
            pltpu.SemaphoreType.DMA,
            pltpu.SemaphoreType.DMA,
            pltpu.SemaphoreType.DMA,
            pltpu.SemaphoreType.DMA,
        ],
    )
    def k(q_hbm, kv_hbm, iq_hbm, ik_hbm, qs_hbm, kvs_hbm,
          iq_v, ik_v, qrows, kvrows, lsem0, lsem1, ssem0, ssem1):
        wid = lax.axis_index("s") * 2 + lax.axis_index("c")
        lsem = (lsem0, lsem1)
        ssem = (ssem0, ssem1)

        def start_load(j, b):
            base = pl.multiple_of((wid * _NCHUNK + j) * _CH, _CH)
            pltpu.make_async_copy(iq_hbm.at[wid, j], iq_v.at[b], lsem[b]).start()
            pltpu.make_async_copy(ik_hbm.at[wid, j], ik_v.at[b], lsem[b]).start()
            pltpu.make_async_copy(q_hbm.at[pl.ds(base, _CH)], qrows.at[b],
                                  lsem[b]).start()
            pltpu.make_async_copy(kv_hbm.at[pl.ds(base, _CH)], kvrows.at[b],
                                  lsem[b]).start()

        def wait_load(j, b):
            base = pl.multiple_of((wid * _NCHUNK + j) * _CH, _CH)
            pltpu.make_async_copy(iq_hbm.at[wid, j], iq_v.at[b], lsem[b]).wait()
            pltpu.make_async_copy(ik_hbm.at[wid, j], ik_v.at[b], lsem[b]).wait()
            pltpu.make_async_copy(q_hbm.at[pl.ds(base, _CH)], qrows.at[b],
                                  lsem[b]).wait()
            pltpu.make_async_copy(kv_hbm.at[pl.ds(base, _CH)], kvrows.at[b],
                                  lsem[b]).wait()

        start_load(0, 0)
        start_load(1, 1)

        def body(jj, _):
            for b in range(2):
                j = jj * 2 + b
                wait_load(j, b)
                cq = pltpu.make_async_copy(qrows.at[b], qs_hbm.at[iq_v.at[b]],
                                           ssem[b])
                ckv = pltpu.make_async_copy(kvrows.at[b],
                                            kvs_hbm.at[ik_v.at[b]], ssem[b])
                cq.start()
                ckv.start()
                cq.wait()
                ckv.wait()

                @pl.when(j + 2 < _NCHUNK)
                def _():
                    start_load(j + 2, b)

            return 0

        lax.fori_loop(0, _NCHUNK // 2, body, 0)

    return k(qf, kvf, idxq, idxk)


def _sc_gather(ocf, idxq):
    nrows = _B * _S * _H
    mesh = plsc.VectorSubcoreMesh(core_axis_name="c", subcore_axis_name="s")

    @functools.partial(
        pl.kernel, mesh=mesh,
        out_type=jax.ShapeDtypeStruct((nrows, 128), jnp.float32),
        scratch_types=[
            pltpu.VMEM((2, _CH), jnp.int32),
            pltpu.VMEM((2, _CH, 128), jnp.float32),
            pltpu.SemaphoreType.DMA,
            pltpu.SemaphoreType.DMA,
            pltpu.SemaphoreType.DMA,
            pltpu.SemaphoreType.DMA,
        ],
    )
    def k(oc_hbm, iq_hbm, out_hbm, iq_v, rows, isem0, isem1, gsem0, gsem1):
        wid = lax.axis_index("s") * 2 + lax.axis_index("c")
        isem = (isem0, isem1)
        gsem = (gsem0, gsem1)

        def start_idx(j, b):
            pltpu.make_async_copy(iq_hbm.at[wid, j], iq_v.at[b], isem[b]).start()

        def wait_idx(j, b):
            pltpu.make_async_copy(iq_hbm.at[wid, j], iq_v.at[b], isem[b]).wait()

        start_idx(0, 0)
        start_idx(1, 1)

        def body(jj, _):
            for b in range(2):
                j = jj * 2 + b
                base = pl.multiple_of((wid * _NCHUNK + j) * _CH, _CH)
                wait_idx(j, b)
                g = pltpu.make_async_copy(oc_hbm.at[iq_v.at[b]], rows.at[b],
                                          gsem[b])
                g.start()
                g.wait()
                s = pltpu.make_async_copy(rows.at[b],
                                          out_hbm.at[pl.ds(base, _CH)],
                                          gsem[b])
                s.start()

                @pl.when(j + 2 < _NCHUNK)
                def _():
                    start_idx(j + 2, b)

                s.wait()

            return 0

        lax.fori_loop(0, _NCHUNK // 2, body, 0)

    return k(ocf, idxq)


def _attn_body(qs_ref, kvs_ref, kvp_ref, o_ref):
    scale = 1.0 / np.sqrt(_DK).astype(np.float32)
    for i in range(_CPB):
        qc = qs_ref[0, 0, i * _BKT:(i + 1) * _BKT, :_DK]
        kvc = kvs_ref[0, 0, i * _BKT:(i + 1) * _BKT, :]
        if i == 0:
            kvp = kvp_ref[0, 0, (_CPB - 1) * _BKT:, :]
        else:
            kvp = kvs_ref[0, 0, (i - 1) * _BKT:i * _BKT, :]
        ke = jnp.concatenate([kvc[:, :_DK], kvp[:, :_DK]], axis=0)
        ve = jnp.concatenate([kvc[:, _DK:], kvp[:, _DK:]], axis=0)
        s = jnp.dot(qc, ke.T, preferred_element_type=jnp.float32) * scale
        m = jnp.max(s, axis=-1, keepdims=True)
        e = jnp.exp(s - m)
        a = e / jnp.sum(e, axis=-1, keepdims=True)
        o = jnp.dot(a, ve, preferred_element_type=jnp.float32)
        o_ref[0, 0, i * _BKT:(i + 1) * _BKT, :] = jnp.concatenate(
            [o, jnp.zeros((_BKT, 128 - _DV), jnp.float32)], axis=1)


def _chunk_attn(qs, kvs):
    grid = (_B, _H, _NCB)
    blk = _CPB * _BKT
    spec = pl.BlockSpec((1, 1, blk, 128), lambda b, h, c: (b, h, c, 0))
    prev = pl.BlockSpec((1, 1, blk, 128),
                        lambda b, h, c: (b, h, (c + _NCB - 1) % _NCB, 0))
    return pl.pallas_call(
        _attn_body,
        grid=grid,
        in_specs=[spec, spec, prev],
        out_specs=spec,
        out_shape=jax.ShapeDtypeStruct((_B, _H, _S, 128), jnp.float32),
    )(qs, kvs, kvs)


def _ln(x, g, b):
    m = jnp.mean(x, axis=-1, keepdims=True)
    xc = x - m
    v = jnp.mean(xc * xc, axis=-1, keepdims=True)
    return xc * jax.lax.rsqrt(v + 1e-6) * g + b


def _proj_ln_body(o_ref, x_ref, wo_ref, g_ref, b_ref, out_ref):
    o = jnp.dot(o_ref[0], wo_ref[...], preferred_element_type=jnp.float32)
    out_ref[0] = _ln(x_ref[0] + o, g_ref[...], b_ref[...])


def _proj_ln(o, x, wo, g, b):
    grid = (_B, _S // _SB)
    dk = o.shape[-1]
    o_spec = pl.BlockSpec((1, _SB, dk), lambda bb, s: (bb, s, 0))
    io_spec = pl.BlockSpec((1, _SB, _D), lambda bb, s: (bb, s, 0))
    return pl.pallas_call(
        _proj_ln_body,
        grid=grid,
        in_specs=[o_spec, io_spec,
                  pl.BlockSpec((dk, _D), lambda bb, s: (0, 0)),
                  pl.BlockSpec((_D,), lambda bb, s: (0,)),
                  pl.BlockSpec((_D,), lambda bb, s: (0,))],
        out_specs=io_spec,
        out_shape=jax.ShapeDtypeStruct((_B, _S, _D), jnp.float32),
    )(o, x, wo, g, b)


def _ffn_body(x_ref, w1_ref, b1_ref, w2_ref, b2_ref, g_ref, b_ref, out_ref):
    x = x_ref[0]
    h = jnp.dot(x, w1_ref[...], preferred_element_type=jnp.float32) + b1_ref[...]
    h = jnp.maximum(h, 0.0)
    y = jnp.dot(h, w2_ref[...], preferred_element_type=jnp.float32) + b2_ref[...]
    out_ref[0] = _ln(x + y, g_ref[...], b_ref[...])


def _ffn(x, p):
    grid = (_B, _S // _SB)
    io_spec = pl.BlockSpec((1, _SB, _D), lambda bb, s: (bb, s, 0))
    return pl.pallas_call(
        _ffn_body,
        grid=grid,
        in_specs=[io_spec,
                  pl.BlockSpec((_D, _EXP * _D), lambda bb, s: (0, 0)),
                  pl.BlockSpec((_EXP * _D,), lambda bb, s: (0,)),
                  pl.BlockSpec((_EXP * _D, _D), lambda bb, s: (0, 0)),
                  pl.BlockSpec((_D,), lambda bb, s: (0,)),
                  pl.BlockSpec((_D,), lambda bb, s: (0,)),
                  pl.BlockSpec((_D,), lambda bb, s: (0,))],
        out_specs=io_spec,
        out_shape=jax.ShapeDtypeStruct((_B, _S, _D), jnp.float32),
    )(x, p['W1'], p['b1'], p['W2'], p['b2'], p['ln_g'], p['ln_b'])


def _lsh_attn(qin, kvin, p, rot):
    qp, kv, bq, bk = _qkv_hash(qin, kvin, p['Wq'], p['Wk'], p['Wv'], rot)
    hq, hk = _hist2(bq, bk)
    dq, dk = _rank2(bq, bk, hq, hk)
    idxq = dq[:, :, :_H].reshape(_NW, _NCHUNK, _CH)
    idxk = dk[:, :, :_H].reshape(_NW, _NCHUNK, _CH)
    qf = qp.reshape(_B * _S * _H, 128)
    kvf = kv.reshape(_B * _S * _H, 128)
    qs, kvs = _sc_scatter2(qf, kvf, idxq, idxk)
    qs = qs.reshape(_B, _H, _S, 128)
    kvs = kvs.reshape(_B, _H, _S, 128)
    oc = _chunk_attn(qs, kvs)
    o = _sc_gather(oc.reshape(_B * _H * _S, 128), idxq)
    o = o.reshape(_B, _S, _H * 128)
    wo_pad = jnp.zeros((_H, 128, _D), jnp.float32)
    wo_pad = wo_pad.at[:, :_DV, :].set(p['Wo'].reshape(_H, _DV, _D))
    wo_pad = wo_pad.reshape(_H * 128, _D)
    return _proj_ln(o, qin, wo_pad, p['ln_g'], p['ln_b'])


def kernel(decoder_input, encoder_output, params):
    rot1 = _rot_const(1)
    rot2 = _rot_const(2)
    x = _lsh_attn(decoder_input, decoder_input, params['self'], rot1)
    x = _lsh_attn(x, encoder_output, params['enc'], rot2)
    return _ffn(x, params['ff'])


# speedup vs baseline: 1.0358x; 1.0003x over previous
import functools
import numpy as np
import jax
import jax.numpy as jnp
from jax import lax
from jax.experimental import pallas as pl
from jax.experimental.pallas import tpu as pltpu
from jax.experimental.pallas import tpu_sc as plsc

_B = 2
_S = 8192
_D = 768
_H = 12
_DK = 64
_DV = 64
_EXP = 4
_BKT = 64
_NC = _S // _BKT
_NHASH = _NC
_SB = 512
_CPB = 16
_NCB = _NC // _CPB


def _rot_const(seed):
    rng = np.random.default_rng(seed)
    nb = _S // _BKT
    r = rng.standard_normal((1, _H, _DK, nb // 2))
    return jnp.asarray(r[0], dtype=jnp.float32)


def _qkv_hash_body(xq_ref, xkv_ref, wq_ref, wkv_ref, rot_ref,
                   q_ref, kv_ref, bq_ref, bk_ref):
    xq = xq_ref[0]
    xkv = xkv_ref[0]
    q = jnp.dot(xq, wq_ref[...], preferred_element_type=jnp.float32)
    kv = jnp.dot(xkv, wkv_ref[...], preferred_element_type=jnp.float32)
    q_ref[0] = q
    kv_ref[0] = kv

    def buckets(mat, off):
        cols = []
        for h in range(_H):
            mh = mat[:, h * 128 + off:h * 128 + off + _DK]
            rq = jnp.dot(mh, rot_ref[h], preferred_element_type=jnp.float32)
            sc = jnp.concatenate([rq, -rq], axis=-1)
            m = jnp.max(sc, axis=-1, keepdims=True)
            lane = jax.lax.broadcasted_iota(jnp.int32, sc.shape, 1)
            idx = jnp.min(jnp.where(sc >= m, lane, _NHASH), axis=-1)
            cols.append(idx.reshape(_SB, 1))
        cols.append(jnp.zeros((_SB, 128 - _H), jnp.int32))
        return jnp.concatenate(cols, axis=-1)

    bq_ref[0] = buckets(q, 0)
    bk_ref[0] = buckets(kv, 0)


def _qkv_hash(xq, xkv, wq, wk, wv, rot):
    nsb = _S // _SB
    grid = (_B, nsb)
    io_spec = pl.BlockSpec((1, _SB, _D), lambda b, s: (b, s, 0))
    w_spec = pl.BlockSpec((_D, _H * 128), lambda b, s: (0, 0))
    p_spec = pl.BlockSpec((1, _SB, _H * 128), lambda b, s: (b, s, 0))
    b_spec = pl.BlockSpec((1, _SB, 128), lambda b, s: (b, s, 0))
    wq_pad = jnp.zeros((_D, _H, 128), jnp.float32)
    wq_pad = wq_pad.at[:, :, :_DK].set(wq.reshape(_D, _H, _DK))
    wkv = jnp.zeros((_D, _H, 128), jnp.float32)
    wkv = wkv.at[:, :, :_DK].set(wk.reshape(_D, _H, _DK))
    wkv = wkv.at[:, :, _DK:].set(wv.reshape(_D, _H, _DV))
    out = pl.pallas_call(
        _qkv_hash_body,
        grid=grid,
        in_specs=[io_spec, io_spec, w_spec, w_spec,
                  pl.BlockSpec((_H, _DK, 64), lambda b, s: (0, 0, 0))],
        out_specs=[p_spec, p_spec, b_spec, b_spec],
        out_shape=[
            jax.ShapeDtypeStruct((_B, _S, _H * 128), jnp.float32),
            jax.ShapeDtypeStruct((_B, _S, _H * 128), jnp.float32),
            jax.ShapeDtypeStruct((_B, _S, 128), jnp.int32),
            jax.ShapeDtypeStruct((_B, _S, 128), jnp.int32),
        ],
    )(xq, xkv, wq_pad.reshape(_D, _H * 128), wkv.reshape(_D, _H * 128), rot)
    return out


_RB = 256
_NW = 32
_CH = 128
_NCHUNK = _B * _S * _H // (_NW * _CH)


def _hist_body(bq_ref, bk_ref, hq_ref, hk_ref):
    s = pl.program_id(1)

    def hist(blk):
        lane = jax.lax.broadcasted_iota(jnp.int32, (_RB, 128), 1)
        rows = []
        for h in range(_H):
            oh = (blk[:, h:h + 1] == lane).astype(jnp.float32)
            rows.append(jnp.sum(oh, axis=0, keepdims=True))
        rows.append(jnp.zeros((16 - _H, 128), jnp.float32))
        return jnp.concatenate(rows, axis=0)

    hq = hist(bq_ref[0])
    hk = hist(bk_ref[0])

    @pl.when(s == 0)
    def _():
        hq_ref[0] = hq
        hk_ref[0] = hk

    @pl.when(s != 0)
    def _():
        hq_ref[0] += hq
        hk_ref[0] += hk


def _hist2(bq, bk):
    grid = (_B, _S // _RB)
    bspec = pl.BlockSpec((1, _RB, 128), lambda b, s: (b, s, 0))
    hspec = pl.BlockSpec((1, 16, 128), lambda b, s: (b, 0, 0))
    return pl.pallas_call(
        _hist_body,
        grid=grid,
        in_specs=[bspec, bspec],
        out_specs=[hspec, hspec],
        out_shape=[jax.ShapeDtypeStruct((_B, 16, 128), jnp.float32)] * 2,
    )(bq, bk)


def _rank_body(bq_ref, bk_ref, hq_ref, hk_ref, dq_ref, dk_ref, cq_ref, ck_ref):
    b = pl.program_id(0)
    s = pl.program_id(1)

    @pl.when(s == 0)
    def _():
        cq_ref[...] = jnp.zeros((16, 128), jnp.float32)
        ck_ref[...] = jnp.zeros((16, 128), jnp.float32)

    i128 = jax.lax.broadcasted_iota(jnp.int32, (128, 128), 0)
    j128 = jax.lax.broadcasted_iota(jnp.int32, (128, 128), 1)
    upper = (i128 < j128).astype(jnp.float32)
    ib = jax.lax.broadcasted_iota(jnp.int32, (_RB, _RB), 0)
    jb = jax.lax.broadcasted_iota(jnp.int32, (_RB, _RB), 1)
    lower = (ib > jb).astype(jnp.bfloat16)
    lane = jax.lax.broadcasted_iota(jnp.int32, (_RB, 128), 1)

    def rank(blk, h_ref, c_ref):
        cum = jnp.dot(h_ref[0], upper, preferred_element_type=jnp.float32)
        cols = []
        for h in range(_H):
            ohb = (blk[:, h:h + 1] == lane).astype(jnp.bfloat16)
            oh = ohb.astype(jnp.float32)
            p = jnp.dot(lower, ohb, preferred_element_type=jnp.float32)
            base = cum[h:h + 1, :] + c_ref[h:h + 1, :]
            pos = jnp.sum((p + base) * oh, axis=1)
            c_ref[h:h + 1, :] += jnp.sum(oh, axis=0, keepdims=True)
            dst = pos.astype(jnp.int32) + (b * _H + h) * _S
            cols.append(dst.reshape(_RB, 1))
        cols.append(jnp.zeros((_RB, 128 - _H), jnp.int32))
        return jnp.concatenate(cols, axis=1)

    dq_ref[0] = rank(bq_ref[0], hq_ref, cq_ref)
    dk_ref[0] = rank(bk_ref[0], hk_ref, ck_ref)


def _rank2(bq, bk, hq, hk):
    grid = (_B, _S // _RB)
    bspec = pl.BlockSpec((1, _RB, 128), lambda b, s: (b, s, 0))
    hspec = pl.BlockSpec((1, 16, 128), lambda b, s: (b, 0, 0))
    return pl.pallas_call(
        _rank_body,
        grid=grid,
        in_specs=[bspec, bspec, hspec, hspec],
        out_specs=[bspec, bspec],
        out_shape=[jax.ShapeDtypeStruct((_B, _S, 128), jnp.int32)] * 2,
        scratch_shapes=[pltpu.VMEM((16, 128), jnp.float32)] * 2,
    )(bq, bk, hq, hk)


def _sc_scatter2(qf, kvf, idxq, idxk):
    nrows = _B * _S * _H
    mesh = plsc.VectorSubcoreMesh(core_axis_name="c", subcore_axis_name="s")

    @functools.partial(
        pl.kernel, mesh=mesh,
        out_type=[jax.ShapeDtypeStruct((nrows, 128), jnp.float32)] * 2,
        scratch_types=[
            pltpu.VMEM((2, _CH), jnp.int32),
            pltpu.VMEM((2, _CH), jnp.int32),
            pltpu.VMEM((2, _CH, 128), jnp.float32),
            pltpu.VMEM((2, _CH, 128), jnp.float32),
            pltpu.SemaphoreType.DMA,
            pltpu.SemaphoreType.DMA,
            pltpu.SemaphoreType.DMA,
            pltpu.SemaphoreType.DMA,
        ],
    )
    def k(q_hbm, kv_hbm, iq_hbm, ik_hbm, qs_hbm, kvs_hbm,
          iq_v, ik_v, qrows, kvrows, lsem0, lsem1, ssem0, ssem1):
        wid = lax.axis_index("s") * 2 + lax.axis_index("c")
        lsem = (lsem0, lsem1)
        ssem = (ssem0, ssem1)

        def start_load(j, b):
            base = pl.multiple_of((wid * _NCHUNK + j) * _CH, _CH)
            pltpu.make_async_copy(iq_hbm.at[wid, j], iq_v.at[b], lsem[b]).start()
            pltpu.make_async_copy(ik_hbm.at[wid, j], ik_v.at[b], lsem[b]).start()
            pltpu.make_async_copy(q_hbm.at[pl.ds(base, _CH)], qrows.at[b],
                                  lsem[b]).start()
            pltpu.make_async_copy(kv_hbm.at[pl.ds(base, _CH)], kvrows.at[b],
                                  lsem[b]).start()

        def wait_load(j, b):
            base = pl.multiple_of((wid * _NCHUNK + j) * _CH, _CH)
            pltpu.make_async_copy(iq_hbm.at[wid, j], iq_v.at[b], lsem[b]).wait()
            pltpu.make_async_copy(ik_hbm.at[wid, j], ik_v.at[b], lsem[b]).wait()
            pltpu.make_async_copy(q_hbm.at[pl.ds(base, _CH)], qrows.at[b],
                                  lsem[b]).wait()
            pltpu.make_async_copy(kv_hbm.at[pl.ds(base, _CH)], kvrows.at[b],
                                  lsem[b]).wait()

        start_load(0, 0)
        start_load(1, 1)

        def body(jj, _):
            for b in range(2):
                j = jj * 2 + b
                wait_load(j, b)
                cq = pltpu.make_async_copy(qrows.at[b], qs_hbm.at[iq_v.at[b]],
                                           ssem[b])
                ckv = pltpu.make_async_copy(kvrows.at[b],
                                            kvs_hbm.at[ik_v.at[b]], ssem[b])
                cq.start()
                ckv.start()
                cq.wait()
                ckv.wait()

                @pl.when(j + 2 < _NCHUNK)
                def _():
                    start_load(j + 2, b)

            return 0

        lax.fori_loop(0, _NCHUNK // 2, body, 0)

    return k(qf, kvf, idxq, idxk)


def _sc_gather(ocf, idxq):
    nrows = _B * _S * _H
    mesh = plsc.VectorSubcoreMesh(core_axis_name="c", subcore_axis_name="s")

    @functools.partial(
        pl.kernel, mesh=mesh,
        out_type=jax.ShapeDtypeStruct((nrows, 128), jnp.float32),
        scratch_types=[
            pltpu.VMEM((2, _CH), jnp.int32),
            pltpu.VMEM((2, _CH, 128), jnp.float32),
            pltpu.SemaphoreType.DMA,
            pltpu.SemaphoreType.DMA,
            pltpu.SemaphoreType.DMA,
            pltpu.SemaphoreType.DMA,
        ],
    )
    def k(oc_hbm, iq_hbm, out_hbm, iq_v, rows, isem0, isem1, gsem0, gsem1):
        wid = lax.axis_index("s") * 2 + lax.axis_index("c")
        isem = (isem0, isem1)
        gsem = (gsem0, gsem1)

        def start_idx(j, b):
            pltpu.make_async_copy(iq_hbm.at[wid, j], iq_v.at[b], isem[b]).start()

        def wait_idx(j, b):
            pltpu.make_async_copy(iq_hbm.at[wid, j], iq_v.at[b], isem[b]).wait()

        start_idx(0, 0)
        start_idx(1, 1)

        def body(jj, _):
            for b in range(2):
                j = jj * 2 + b
                base = pl.multiple_of((wid * _NCHUNK + j) * _CH, _CH)
                wait_idx(j, b)
                g = pltpu.make_async_copy(oc_hbm.at[iq_v.at[b]], rows.at[b],
                                          gsem[b])
                g.start()
                g.wait()
                s = pltpu.make_async_copy(rows.at[b],
                                          out_hbm.at[pl.ds(base, _CH)],
                                          gsem[b])
                s.start()

                @pl.when(j + 2 < _NCHUNK)
                def _():
                    start_idx(j + 2, b)

                s.wait()

            return 0

        lax.fori_loop(0, _NCHUNK // 2, body, 0)

    return k(ocf, idxq)


def _attn_body(qs_ref, kvs_ref, kvp_ref, o_ref):
    scale = 1.0 / np.sqrt(_DK).astype(np.float32)
    for i in range(_CPB):
        qc = qs_ref[0, 0, i * _BKT:(i + 1) * _BKT, :_DK]
        kvc = kvs_ref[0, 0, i * _BKT:(i + 1) * _BKT, :]
        if i == 0:
            kvp = kvp_ref[0, 0, (_CPB - 1) * _BKT:, :]
        else:
            kvp = kvs_ref[0, 0, (i - 1) * _BKT:i * _BKT, :]
        ke = jnp.concatenate([kvc[:, :_DK], kvp[:, :_DK]], axis=0)
        ve = jnp.concatenate([kvc[:, _DK:], kvp[:, _DK:]], axis=0)
        s = jnp.dot(qc, ke.T, preferred_element_type=jnp.float32) * scale
        m = jnp.max(s, axis=-1, keepdims=True)
        e = jnp.exp(s - m)
        a = e / jnp.sum(e, axis=-1, keepdims=True)
        o = jnp.dot(a, ve, preferred_element_type=jnp.float32)
        o_ref[0, 0, i * _BKT:(i + 1) * _BKT, :] = jnp.concatenate(
            [o, jnp.zeros((_BKT, 128 - _DV), jnp.float32)], axis=1)


def _chunk_attn(qs, kvs):
    grid = (_B, _H, _NCB)
    blk = _CPB * _BKT
    spec = pl.BlockSpec((1, 1, blk, 128), lambda b, h, c: (b, h, c, 0))
    prev = pl.BlockSpec((1, 1, blk, 128),
                        lambda b, h, c: (b, h, (c + _NCB - 1) % _NCB, 0))
    return pl.pallas_call(
        _attn_body,
        grid=grid,
        in_specs=[spec, spec, prev],
        out_specs=spec,
        out_shape=jax.ShapeDtypeStruct((_B, _H, _S, 128), jnp.float32),
    )(qs, kvs, kvs)


def _ln(x, g, b):
    m = jnp.mean(x, axis=-1, keepdims=True)
    xc = x - m
    v = jnp.mean(xc * xc, axis=-1, keepdims=True)
    return xc * jax.lax.rsqrt(v + 1e-6) * g + b


def _proj_ln_body(low, o_ref, x_ref, wo_ref, g_ref, b_ref, out_ref):
    o = o_ref[0]
    w = wo_ref[...]
    if low:
        o = o.astype(jnp.bfloat16)
        w = w.astype(jnp.bfloat16)
    o = jnp.dot(o, w, preferred_element_type=jnp.float32)
    out_ref[0] = _ln(x_ref[0] + o, g_ref[...], b_ref[...])


def _proj_ln(o, x, wo, g, b, low=False):
    grid = (_B, _S // _SB)
    dk = o.shape[-1]
    o_spec = pl.BlockSpec((1, _SB, dk), lambda bb, s: (bb, s, 0))
    io_spec = pl.BlockSpec((1, _SB, _D), lambda bb, s: (bb, s, 0))
    return pl.pallas_call(
        functools.partial(_proj_ln_body, low),
        grid=grid,
        in_specs=[o_spec, io_spec,
                  pl.BlockSpec((dk, _D), lambda bb, s: (0, 0)),
                  pl.BlockSpec((_D,), lambda bb, s: (0,)),
                  pl.BlockSpec((_D,), lambda bb, s: (0,))],
        out_specs=io_spec,
        out_shape=jax.ShapeDtypeStruct((_B, _S, _D), jnp.float32),
    )(o, x, wo, g, b)


def _ffn_body(x_ref, w1_ref, b1_ref, w2_ref, b2_ref, g_ref, b_ref, out_ref):
    x = x_ref[0]
    h = jnp.dot(x.astype(jnp.bfloat16), w1_ref[...].astype(jnp.bfloat16),
                preferred_element_type=jnp.float32) + b1_ref[...]
    h = jnp.maximum(h, 0.0)
    y = jnp.dot(h.astype(jnp.bfloat16), w2_ref[...].astype(jnp.bfloat16),
                preferred_element_type=jnp.float32) + b2_ref[...]
    out_ref[0] = _ln(x + y, g_ref[...], b_ref[...])


def _ffn(x, p):
    grid = (_B, _S // _SB)
    io_spec = pl.BlockSpec((1, _SB, _D), lambda bb, s: (bb, s, 0))
    return pl.pallas_call(
        _ffn_body,
        grid=grid,
        in_specs=[io_spec,
                  pl.BlockSpec((_D, _EXP * _D), lambda bb, s: (0, 0)),
                  pl.BlockSpec((_EXP * _D,), lambda bb, s: (0,)),
                  pl.BlockSpec((_EXP * _D, _D), lambda bb, s: (0, 0)),
                  pl.BlockSpec((_D,), lambda bb, s: (0,)),
                  pl.BlockSpec((_D,), lambda bb, s: (0,)),
                  pl.BlockSpec((_D,), lambda bb, s: (0,))],
        out_specs=io_spec,
        out_shape=jax.ShapeDtypeStruct((_B, _S, _D), jnp.float32),
    )(x, p['W1'], p['b1'], p['W2'], p['b2'], p['ln_g'], p['ln_b'])


def _lsh_attn(qin, kvin, p, rot, low=False):
    qp, kv, bq, bk = _qkv_hash(qin, kvin, p['Wq'], p['Wk'], p['Wv'], rot)
    hq, hk = _hist2(bq, bk)
    dq, dk = _rank2(bq, bk, hq, hk)
    idxq = dq[:, :, :_H].reshape(_NW, _NCHUNK, _CH)
    idxk = dk[:, :, :_H].reshape(_NW, _NCHUNK, _CH)
    qf = qp.reshape(_B * _S * _H, 128)
    kvf = kv.reshape(_B * _S * _H, 128)
    qs, kvs = _sc_scatter2(qf, kvf, idxq, idxk)
    qs = qs.reshape(_B, _H, _S, 128)
    kvs = kvs.reshape(_B, _H, _S, 128)
    oc = _chunk_attn(qs, kvs)
    o = _sc_gather(oc.reshape(_B * _H * _S, 128), idxq)
    o = o.reshape(_B, _S, _H * 128)
    wo_pad = jnp.zeros((_H, 128, _D), jnp.float32)
    wo_pad = wo_pad.at[:, :_DV, :].set(p['Wo'].reshape(_H, _DV, _D))
    wo_pad = wo_pad.reshape(_H * 128, _D)
    return _proj_ln(o, qin, wo_pad, p['ln_g'], p['ln_b'], low=low)


def kernel(decoder_input, encoder_output, params):
    rot1 = _rot_const(1)
    rot2 = _rot_const(2)
    x = _lsh_attn(decoder_input, decoder_input, params['self'], rot1)
    x = _lsh_attn(x, encoder_output, params['enc'], rot2, low=True)
    return _ffn(x, params['ff'])
